# baseline (speedup 1.0000x reference)
import functools

import jax
import jax.numpy as jnp
from jax import lax
from jax.experimental import pallas as pl
from jax.experimental.pallas import tpu as pltpu
from jax.experimental.pallas import tpu_sc as plsc

N = 10000
E = 320000
D = 128

NC = 2
NS = 16
NW = NC * NS
NPAD = 10240
EP = E // NW
CH = 64
NB = 2
NPH = 2
NCH = (-(-EP // CH) + NB * NPH - 1) // (NB * NPH) * (NB * NPH)
NCHP = NCH // NPH
EPP = NCH * CH
RPS = NPAD // NS

_MESH = plsc.VectorSubcoreMesh(
    core_axis_name="c", subcore_axis_name="s", num_cores=NC, num_subcores=NS
)


def _deg_body(sidx, didx, degp, sv, dv, ones_v, zv, d0, d1):
    c = lax.axis_index("c")
    s = lax.axis_index("s")
    wid = s * NC + c

    def _ones(i, _):
        ones_v[pl.ds(i * 16, 16)] = jnp.ones((16,), jnp.float32)
        return 0

    lax.fori_loop(0, CH // 16, _ones, 0)

    def _zeros(i, _):
        zv[pl.ds(i * 16, 16)] = jnp.zeros((16,), jnp.float32)
        return 0

    lax.fori_loop(0, RPS // 16, _zeros, 0)
    pltpu.sync_copy(zv, d0.at[pl.ds(s * RPS, RPS)])
    pltpu.sync_copy(zv, d1.at[pl.ds(s * RPS, RPS)])
    plsc.subcore_barrier()

    def _scat(j, _):
        pltpu.sync_copy(ones_v, d0.at[sv.at[j]], add=True)
        pltpu.sync_copy(ones_v, d1.at[dv.at[j]], add=True)
        return 0

    for p in range(NPH):
        pltpu.sync_copy(sidx.at[wid, pl.ds(p * NCHP, NCHP)], sv)
        pltpu.sync_copy(didx.at[wid, pl.ds(p * NCHP, NCHP)], dv)
        lax.fori_loop(0, NCHP, _scat, 0)
    plsc.subcore_barrier()
    pltpu.sync_copy(d0.at[pl.ds(s * RPS, RPS)], degp.at[c, 0, pl.ds(s * RPS, RPS)])
    pltpu.sync_copy(d1.at[pl.ds(s * RPS, RPS)], degp.at[c, 1, pl.ds(s * RPS, RPS)])


_deg_call = functools.partial(
    pl.kernel,
    out_type=jax.ShapeDtypeStruct((NC, 2, NPAD), jnp.float32),
    mesh=_MESH,
    scratch_types=[
        pltpu.VMEM((NCHP, CH), jnp.int32),
        pltpu.VMEM((NCHP, CH), jnp.int32),
        pltpu.VMEM((CH,), jnp.float32),
        pltpu.VMEM((RPS,), jnp.float32),
        pltpu.VMEM_SHARED((NPAD,), jnp.float32),
        pltpu.VMEM_SHARED((NPAD,), jnp.float32),
    ],
)(_deg_body)


def _scatter_body(h, sidx, didx, part, sv, dv, rows, acc, *sems):
    gsem = sems[:NB]
    ssem = sems[NB:]
    c = lax.axis_index("c")
    s = lax.axis_index("s")
    wid = s * NC + c

    def _zb(i, _):
        for k in range(D // 16):
            rows[0, i, pl.ds(k * 16, 16)] = jnp.zeros((16,), jnp.float32)
        return 0

    lax.fori_loop(0, CH, _zb, 0)
    for k in range(RPS // CH):
        pltpu.sync_copy(rows.at[0], acc.at[pl.ds(s * RPS + k * CH, CH)])
    plsc.subcore_barrier()

    def _group(g, _):
        for b in range(NB):
            j = g * NB + b

            @pl.when(g > 0)
            def _drain():
                pltpu.make_async_copy(
                    rows.at[b], acc.at[dv.at[j - NB]], ssem[b]
                ).wait()

            pltpu.async_copy(h.at[sv.at[j]], rows.at[b], gsem[b])
        for b in range(NB):
            j = g * NB + b
            pltpu.make_async_copy(h.at[sv.at[j]], rows.at[b], gsem[b]).wait()
            pltpu.async_copy(rows.at[b], acc.at[dv.at[j]], ssem[b], add=True)
        return 0

    for p in range(NPH):
        pltpu.sync_copy(sidx.at[wid, pl.ds(p * NCHP, NCHP)], sv)
        pltpu.sync_copy(didx.at[wid, pl.ds(p * NCHP, NCHP)], dv)
        lax.fori_loop(0, NCHP // NB, _group, 0)
        for b in range(NB):
            pltpu.make_async_copy(
                rows.at[b], acc.at[dv.at[NCHP - NB + b]], ssem[b]
            ).wait()
    plsc.subcore_barrier()
    pltpu.sync_copy(acc.at[pl.ds(s * RPS, RPS)], part.at[c, pl.ds(s * RPS, RPS)])


_scatter_call = functools.partial(
    pl.kernel,
    out_type=jax.ShapeDtypeStruct((NC, NPAD, D), jnp.float32),
    mesh=_MESH,
    scratch_types=[
        pltpu.VMEM((NCHP, CH), jnp.int32),
        pltpu.VMEM((NCHP, CH), jnp.int32),
        pltpu.VMEM((NB, CH, D), jnp.float32),
        pltpu.VMEM_SHARED((NPAD, D), jnp.float32),
    ]
    + [pltpu.SemaphoreType.DMA] * (2 * NB),
)(_scatter_body)


def _mm_body(x_ref, w_ref, b_ref, deg_ref, o_ref):
    h = jnp.dot(x_ref[...], w_ref[...], preferred_element_type=jnp.float32)
    h = h + b_ref[...]
    dsum = deg_ref[0, :] + deg_ref[2, :]
    o_ref[...] = h * lax.rsqrt(jnp.clip(dsum, 1.0, None))[:, None]


_BN1 = 1024
_mm_call = pl.pallas_call(
    _mm_body,
    grid=(NPAD // _BN1,),
    in_specs=[
        pl.BlockSpec((_BN1, D), lambda j: (j, 0)),
        pl.BlockSpec((D, D), lambda j: (0, 0)),
        pl.BlockSpec((1, D), lambda j: (0, 0)),
        pl.BlockSpec((2 * NC, _BN1), lambda j: (0, j)),
    ],
    out_specs=pl.BlockSpec((_BN1, D), lambda j: (j, 0)),
    out_shape=jax.ShapeDtypeStruct((NPAD, D), jnp.float32),
)


def _comb_body(p_ref, deg_ref, o_ref):
    ssum = p_ref[0] + p_ref[1]
    dsum = deg_ref[1, :] + deg_ref[3, :]
    o_ref[...] = ssum * lax.rsqrt(jnp.clip(dsum, 1.0, None))[:, None]


_BN2 = 1024
_comb_call = pl.pallas_call(
    _comb_body,
    grid=(NPAD // _BN2,),
    in_specs=[
        pl.BlockSpec((NC, _BN2, D), lambda j: (0, j, 0)),
        pl.BlockSpec((2 * NC, _BN2), lambda j: (0, j)),
    ],
    out_specs=pl.BlockSpec((_BN2, D), lambda j: (j, 0)),
    out_shape=jax.ShapeDtypeStruct((NPAD, D), jnp.float32),
)


def kernel(features, edge_index, W, b):
    src = edge_index[0].reshape(NW, EP)
    dst = edge_index[1].reshape(NW, EP)
    pad = N + (jnp.arange(EPP - EP, dtype=jnp.int32) % (NPAD - N))
    pad = jnp.broadcast_to(pad[None, :], (NW, EPP - EP))
    sidx = jnp.concatenate([src, pad], axis=1).reshape(NW, NCH, CH)
    didx = jnp.concatenate([dst, pad], axis=1).reshape(NW, NCH, CH)
    xp = jnp.concatenate(
        [features, jnp.zeros((NPAD - N, D), jnp.float32)], axis=0
    )

    degp = _deg_call(sidx, didx).reshape(2 * NC, NPAD)
    hs = _mm_call(xp, W, b.reshape(1, D), degp)
    part = _scatter_call(hs, sidx, didx)
    return _comb_call(part, degp)[:N]


# speedup vs baseline: 1.1741x; 1.1741x over previous
import functools

import jax
import jax.numpy as jnp
from jax import lax
from jax.experimental import pallas as pl
from jax.experimental.pallas import tpu as pltpu
from jax.experimental.pallas import tpu_sc as plsc

N = 10000
E = 320000
D = 128

NC = 2
NS = 16
NW = NC * NS
NPAD = 10240
EP = E // NW
CH = 128
NB = 2
NCH = 80
EPP = NCH * CH
NPH = 8
NCHP = NCH // NPH
NPHD = 8
NCHD = NCH // NPHD
RPS = NPAD // NS

_MESH = plsc.VectorSubcoreMesh(
    core_axis_name="c", subcore_axis_name="s", num_cores=NC, num_subcores=NS
)


def _deg_body(sidx, didx, degp, sv, dv, ones_v, zv, d0, d1, sem0, sem1, isem):
    c = lax.axis_index("c")
    s = lax.axis_index("s")
    wid = s * NC + c

    def _ones(i, _):
        ones_v[pl.ds(i * 16, 16)] = jnp.ones((16,), jnp.float32)
        return 0

    lax.fori_loop(0, CH // 16, _ones, 0)

    def _zeros(i, _):
        zv[pl.ds(i * 16, 16)] = jnp.zeros((16,), jnp.float32)
        return 0

    lax.fori_loop(0, RPS // 16, _zeros, 0)
    pltpu.sync_copy(zv, d0.at[pl.ds(s * RPS, RPS)])
    pltpu.sync_copy(zv, d1.at[pl.ds(s * RPS, RPS)])
    pltpu.sync_copy(sidx.at[wid, 0], sv.at[0])
    pltpu.sync_copy(didx.at[wid, 0], dv.at[0])
    plsc.subcore_barrier()

    for p in range(NPHD):
        if p + 1 < NPHD:
            pltpu.async_copy(sidx.at[wid, p + 1], sv.at[(p + 1) % 2], isem)
            pltpu.async_copy(didx.at[wid, p + 1], dv.at[(p + 1) % 2], isem)
        svp = sv.at[p % 2]
        dvp = dv.at[p % 2]

        def _scat(j, _, svp=svp, dvp=dvp, first=(p == 0)):
            def _drain():
                pltpu.make_async_copy(ones_v, d0.at[svp.at[0]], sem0).wait()
                pltpu.make_async_copy(ones_v, d1.at[dvp.at[0]], sem1).wait()

            if first:
                pl.when(j > 0)(_drain)
            else:
                _drain()
            pltpu.async_copy(ones_v, d0.at[svp.at[j]], sem0, add=True)
            pltpu.async_copy(ones_v, d1.at[dvp.at[j]], sem1, add=True)
            return 0

        lax.fori_loop(0, NCHD, _scat, 0)
        if p + 1 < NPHD:
            pltpu.make_async_copy(
                sidx.at[wid, p + 1], sv.at[(p + 1) % 2], isem
            ).wait()
            pltpu.make_async_copy(
                didx.at[wid, p + 1], dv.at[(p + 1) % 2], isem
            ).wait()
    pltpu.make_async_copy(ones_v, d0.at[sv.at[0, 0]], sem0).wait()
    pltpu.make_async_copy(ones_v, d1.at[dv.at[0, 0]], sem1).wait()
    plsc.subcore_barrier()
    pltpu.sync_copy(d0.at[pl.ds(s * RPS, RPS)], degp.at[c, 0, pl.ds(s * RPS, RPS)])
    pltpu.sync_copy(d1.at[pl.ds(s * RPS, RPS)], degp.at[c, 1, pl.ds(s * RPS, RPS)])


_deg_call = functools.partial(
    pl.kernel,
    out_type=jax.ShapeDtypeStruct((NC, 2, NPAD), jnp.float32),
    mesh=_MESH,
    scratch_types=[
        pltpu.VMEM((2, NCHD, CH), jnp.int32),
        pltpu.VMEM((2, NCHD, CH), jnp.int32),
        pltpu.VMEM((CH,), jnp.float32),
        pltpu.VMEM((RPS,), jnp.float32),
        pltpu.VMEM_SHARED((NPAD,), jnp.float32),
        pltpu.VMEM_SHARED((NPAD,), jnp.float32),
        pltpu.SemaphoreType.DMA,
        pltpu.SemaphoreType.DMA,
        pltpu.SemaphoreType.DMA,
    ],
)(_deg_body)


def _scatter_body(h, sidx, didx, part, svb, dvb, rows, acc, *sems):
    gsem = sems[:NB]
    ssem = sems[NB : 2 * NB]
    isem = sems[2 * NB]
    c = lax.axis_index("c")
    s = lax.axis_index("s")
    wid = s * NC + c

    def _zb(i, _):
        for k in range(D // 16):
            rows[0, i, pl.ds(k * 16, 16)] = jnp.zeros((16,), jnp.float32)
        return 0

    lax.fori_loop(0, CH, _zb, 0)
    pltpu.sync_copy(sidx.at[wid, 0], svb.at[0])
    pltpu.sync_copy(didx.at[wid, 0], dvb.at[0])
    for k in range(RPS // CH):
        pltpu.sync_copy(rows.at[0], acc.at[pl.ds(s * RPS + k * CH, CH)])
    plsc.subcore_barrier()

    for p in range(NPH):
        if p + 1 < NPH:
            pltpu.async_copy(sidx.at[wid, p + 1], svb.at[(p + 1) % 2], isem)
            pltpu.async_copy(didx.at[wid, p + 1], dvb.at[(p + 1) % 2], isem)
        svp = svb.at[p % 2]
        dvp = dvb.at[p % 2]

        def _group(g, _, svp=svp, dvp=dvp, first=(p == 0)):
            for b in range(NB):

                def _drain(b=b, dvp=dvp):
                    pltpu.make_async_copy(
                        rows.at[b], acc.at[dvp.at[0]], ssem[b]
                    ).wait()

                if first:
                    pl.when(g > 0)(_drain)
                else:
                    _drain()
                pltpu.async_copy(h.at[svp.at[g * NB + b]], rows.at[b], gsem[b])
            for b in range(NB):
                pltpu.make_async_copy(h.at[svp.at[0]], rows.at[b], gsem[b]).wait()
                pltpu.async_copy(
                    rows.at[b], acc.at[dvp.at[g * NB + b]], ssem[b], add=True
                )
            return 0

        lax.fori_loop(0, NCHP // NB, _group, 0)
        if p + 1 < NPH:
            pltpu.make_async_copy(
                sidx.at[wid, p + 1], svb.at[(p + 1) % 2], isem
            ).wait()
            pltpu.make_async_copy(
                didx.at[wid, p + 1], dvb.at[(p + 1) % 2], isem
            ).wait()
    for b in range(NB):
        pltpu.make_async_copy(rows.at[b], acc.at[dvb.at[0, 0]], ssem[b]).wait()
    plsc.subcore_barrier()
    pltpu.sync_copy(acc.at[pl.ds(s * RPS, RPS)], part.at[c, pl.ds(s * RPS, RPS)])


_scatter_call = functools.partial(
    pl.kernel,
    out_type=jax.ShapeDtypeStruct((NC, NPAD, D), jnp.float32),
    mesh=_MESH,
    scratch_types=[
        pltpu.VMEM((2, NCHP, CH), jnp.int32),
        pltpu.VMEM((2, NCHP, CH), jnp.int32),
        pltpu.VMEM((NB, CH, D), jnp.float32),
        pltpu.VMEM_SHARED((NPAD, D), jnp.float32),
    ]
    + [pltpu.SemaphoreType.DMA] * (2 * NB + 1),
)(_scatter_body)


def _mm_body(x_ref, w_ref, b_ref, deg_ref, o_ref):
    h = jnp.dot(x_ref[...], w_ref[...], preferred_element_type=jnp.float32)
    h = h + b_ref[...]
    dsum = deg_ref[0, :] + deg_ref[2, :]
    o_ref[...] = h * lax.rsqrt(jnp.clip(dsum, 1.0, None))[:, None]


_BN1 = 1024
_mm_call = pl.pallas_call(
    _mm_body,
    grid=(NPAD // _BN1,),
    in_specs=[
        pl.BlockSpec((_BN1, D), lambda j: (j, 0)),
        pl.BlockSpec((D, D), lambda j: (0, 0)),
        pl.BlockSpec((1, D), lambda j: (0, 0)),
        pl.BlockSpec((2 * NC, _BN1), lambda j: (0, j)),
    ],
    out_specs=pl.BlockSpec((_BN1, D), lambda j: (j, 0)),
    out_shape=jax.ShapeDtypeStruct((NPAD, D), jnp.float32),
)


def _comb_body(p_ref, deg_ref, o_ref):
    ssum = p_ref[0] + p_ref[1]
    dsum = deg_ref[1, :] + deg_ref[3, :]
    o_ref[...] = ssum * lax.rsqrt(jnp.clip(dsum, 1.0, None))[:, None]


_BN2 = 1024
_comb_call = pl.pallas_call(
    _comb_body,
    grid=(NPAD // _BN2,),
    in_specs=[
        pl.BlockSpec((NC, _BN2, D), lambda j: (0, j, 0)),
        pl.BlockSpec((2 * NC, _BN2), lambda j: (0, j)),
    ],
    out_specs=pl.BlockSpec((_BN2, D), lambda j: (j, 0)),
    out_shape=jax.ShapeDtypeStruct((NPAD, D), jnp.float32),
)


def kernel(features, edge_index, W, b):
    src = edge_index[0].reshape(NW, EP)
    dst = edge_index[1].reshape(NW, EP)
    pad = N + (jnp.arange(EPP - EP, dtype=jnp.int32) % (NPAD - N))
    pad = jnp.broadcast_to(pad[None, :], (NW, EPP - EP))
    sidx = jnp.concatenate([src, pad], axis=1).reshape(NW, NCH, CH)
    didx = jnp.concatenate([dst, pad], axis=1).reshape(NW, NCH, CH)
    xp = jnp.concatenate(
        [features, jnp.zeros((NPAD - N, D), jnp.float32)], axis=0
    )

    sidx_d = sidx.reshape(NW, NPHD, NCHD, CH)
    didx_d = didx.reshape(NW, NPHD, NCHD, CH)
    sidx_s = sidx.reshape(NW, NPH, NCHP, CH)
    didx_s = didx.reshape(NW, NPH, NCHP, CH)
    degp = _deg_call(sidx_d, didx_d).reshape(2 * NC, NPAD)
    hs = _mm_call(xp, W, b.reshape(1, D), degp)
    part = _scatter_call(hs, sidx_s, didx_s)
    return _comb_call(part, degp)[:N]


# speedup vs baseline: 1.3094x; 1.1153x over previous
import functools

import jax
import jax.numpy as jnp
from jax import lax
from jax.experimental import pallas as pl
from jax.experimental.pallas import tpu as pltpu
from jax.experimental.pallas import tpu_sc as plsc

N = 10000
E = 320000
D = 128

NC = 2
NS = 16
NW = NC * NS
NPAD = 10240
EP = E // NW
CH = 64
NB = 4
NCH = 160
EPP = NCH * CH
NPH = 8
NCHP = NCH // NPH
NPHD = 8
NCHD = NCH // NPHD
RPS = NPAD // NS

_MESH = plsc.VectorSubcoreMesh(
    core_axis_name="c", subcore_axis_name="s", num_cores=NC, num_subcores=NS
)


def _deg_body(sidx, didx, degp, sv, dv, ones_v, zv, d0, d1, sem0, sem1, isem):
    c = lax.axis_index("c")
    s = lax.axis_index("s")
    wid = s * NC + c

    def _ones(i, _):
        ones_v[pl.ds(i * 16, 16)] = jnp.ones((16,), jnp.float32)
        return 0

    lax.fori_loop(0, CH // 16, _ones, 0)

    def _zeros(i, _):
        zv[pl.ds(i * 16, 16)] = jnp.zeros((16,), jnp.float32)
        return 0

    lax.fori_loop(0, RPS // 16, _zeros, 0)
    pltpu.sync_copy(zv, d0.at[pl.ds(s * RPS, RPS)])
    pltpu.sync_copy(zv, d1.at[pl.ds(s * RPS, RPS)])
    pltpu.sync_copy(sidx.at[wid, 0], sv.at[0])
    pltpu.sync_copy(didx.at[wid, 0], dv.at[0])
    plsc.subcore_barrier()

    for p in range(NPHD):
        if p + 1 < NPHD:
            pltpu.async_copy(sidx.at[wid, p + 1], sv.at[(p + 1) % 2], isem)
            pltpu.async_copy(didx.at[wid, p + 1], dv.at[(p + 1) % 2], isem)
        svp = sv.at[p % 2]
        dvp = dv.at[p % 2]

        def _scat(j, _, svp=svp, dvp=dvp, first=(p == 0)):
            def _drain():
                pltpu.make_async_copy(ones_v, d0.at[svp.at[0]], sem0).wait()
                pltpu.make_async_copy(ones_v, d1.at[dvp.at[0]], sem1).wait()

            if first:
                pl.when(j > 0)(_drain)
            else:
                _drain()
            pltpu.async_copy(ones_v, d0.at[svp.at[j]], sem0, add=True)
            pltpu.async_copy(ones_v, d1.at[dvp.at[j]], sem1, add=True)
            return 0

        lax.fori_loop(0, NCHD, _scat, 0)
        if p + 1 < NPHD:
            pltpu.make_async_copy(
                sidx.at[wid, p + 1], sv.at[(p + 1) % 2], isem
            ).wait()
            pltpu.make_async_copy(
                didx.at[wid, p + 1], dv.at[(p + 1) % 2], isem
            ).wait()
    pltpu.make_async_copy(ones_v, d0.at[sv.at[0, 0]], sem0).wait()
    pltpu.make_async_copy(ones_v, d1.at[dv.at[0, 0]], sem1).wait()
    plsc.subcore_barrier()
    pltpu.sync_copy(d0.at[pl.ds(s * RPS, RPS)], degp.at[c, 0, pl.ds(s * RPS, RPS)])
    pltpu.sync_copy(d1.at[pl.ds(s * RPS, RPS)], degp.at[c, 1, pl.ds(s * RPS, RPS)])


_deg_call = functools.partial(
    pl.kernel,
    out_type=jax.ShapeDtypeStruct((NC, 2, NPAD), jnp.float32),
    mesh=_MESH,
    scratch_types=[
        pltpu.VMEM((2, NCHD, CH), jnp.int32),
        pltpu.VMEM((2, NCHD, CH), jnp.int32),
        pltpu.VMEM((CH,), jnp.float32),
        pltpu.VMEM((RPS,), jnp.float32),
        pltpu.VMEM_SHARED((NPAD,), jnp.float32),
        pltpu.VMEM_SHARED((NPAD,), jnp.float32),
        pltpu.SemaphoreType.DMA,
        pltpu.SemaphoreType.DMA,
        pltpu.SemaphoreType.DMA,
    ],
)(_deg_body)


def _scatter_body(h, sidx, didx, part, svb, dvb, rows, acc, *sems):
    gsem = sems[:NB]
    ssem = sems[NB : 2 * NB]
    isem = sems[2 * NB]
    c = lax.axis_index("c")
    s = lax.axis_index("s")
    wid = s * NC + c

    def _zb(i, _):
        for k in range(D // 16):
            rows[0, i, pl.ds(k * 16, 16)] = jnp.zeros((16,), jnp.float32)
        return 0

    lax.fori_loop(0, CH, _zb, 0)
    pltpu.sync_copy(sidx.at[wid, 0], svb.at[0])
    pltpu.sync_copy(didx.at[wid, 0], dvb.at[0])
    for k in range(RPS // CH):
        pltpu.sync_copy(rows.at[0], acc.at[pl.ds(s * RPS + k * CH, CH)])
    plsc.subcore_barrier()

    for p in range(NPH):
        if p + 1 < NPH:
            pltpu.async_copy(sidx.at[wid, p + 1], svb.at[(p + 1) % 2], isem)
            pltpu.async_copy(didx.at[wid, p + 1], dvb.at[(p + 1) % 2], isem)
        svp = svb.at[p % 2]
        dvp = dvb.at[p % 2]

        def _group(g, _, svp=svp, dvp=dvp, first=(p == 0)):
            for b in range(NB):

                def _drain(b=b, dvp=dvp):
                    pltpu.make_async_copy(
                        rows.at[b], acc.at[dvp.at[0]], ssem[b]
                    ).wait()

                if first:
                    pl.when(g > 0)(_drain)
                else:
                    _drain()
                pltpu.async_copy(h.at[svp.at[g * NB + b]], rows.at[b], gsem[b])
            for b in range(NB):
                pltpu.make_async_copy(h.at[svp.at[0]], rows.at[b], gsem[b]).wait()
                pltpu.async_copy(
                    rows.at[b], acc.at[dvp.at[g * NB + b]], ssem[b], add=True
                )
            return 0

        lax.fori_loop(0, NCHP // NB, _group, 0)
        if p + 1 < NPH:
            pltpu.make_async_copy(
                sidx.at[wid, p + 1], svb.at[(p + 1) % 2], isem
            ).wait()
            pltpu.make_async_copy(
                didx.at[wid, p + 1], dvb.at[(p + 1) % 2], isem
            ).wait()
    for b in range(NB):
        pltpu.make_async_copy(rows.at[b], acc.at[dvb.at[0, 0]], ssem[b]).wait()
    plsc.subcore_barrier()
    pltpu.sync_copy(acc.at[pl.ds(s * RPS, RPS)], part.at[c, pl.ds(s * RPS, RPS)])


_scatter_call = functools.partial(
    pl.kernel,
    out_type=jax.ShapeDtypeStruct((NC, NPAD, D), jnp.float32),
    mesh=_MESH,
    scratch_types=[
        pltpu.VMEM((2, NCHP, CH), jnp.int32),
        pltpu.VMEM((2, NCHP, CH), jnp.int32),
        pltpu.VMEM((NB, CH, D), jnp.float32),
        pltpu.VMEM_SHARED((NPAD, D), jnp.float32),
    ]
    + [pltpu.SemaphoreType.DMA] * (2 * NB + 1),
)(_scatter_body)


def _mm_body(x_ref, w_ref, b_ref, deg_ref, o_ref):
    h = jnp.dot(x_ref[...], w_ref[...], preferred_element_type=jnp.float32)
    h = h + b_ref[...]
    dsum = deg_ref[0, :] + deg_ref[2, :]
    o_ref[...] = h * lax.rsqrt(jnp.clip(dsum, 1.0, None))[:, None]


_BN1 = 1024
_mm_call = pl.pallas_call(
    _mm_body,
    grid=(NPAD // _BN1,),
    in_specs=[
        pl.BlockSpec((_BN1, D), lambda j: (j, 0)),
        pl.BlockSpec((D, D), lambda j: (0, 0)),
        pl.BlockSpec((1, D), lambda j: (0, 0)),
        pl.BlockSpec((2 * NC, _BN1), lambda j: (0, j)),
    ],
    out_specs=pl.BlockSpec((_BN1, D), lambda j: (j, 0)),
    out_shape=jax.ShapeDtypeStruct((NPAD, D), jnp.float32),
)


def _comb_body(p_ref, deg_ref, o_ref):
    ssum = p_ref[0] + p_ref[1]
    dsum = deg_ref[1, :] + deg_ref[3, :]
    o_ref[...] = ssum * lax.rsqrt(jnp.clip(dsum, 1.0, None))[:, None]


_BN2 = 1024
_comb_call = pl.pallas_call(
    _comb_body,
    grid=(NPAD // _BN2,),
    in_specs=[
        pl.BlockSpec((NC, _BN2, D), lambda j: (0, j, 0)),
        pl.BlockSpec((2 * NC, _BN2), lambda j: (0, j)),
    ],
    out_specs=pl.BlockSpec((_BN2, D), lambda j: (j, 0)),
    out_shape=jax.ShapeDtypeStruct((NPAD, D), jnp.float32),
)


def kernel(features, edge_index, W, b):
    src = edge_index[0].reshape(NW, EP)
    dst = edge_index[1].reshape(NW, EP)
    pad = N + (jnp.arange(EPP - EP, dtype=jnp.int32) % (NPAD - N))
    pad = jnp.broadcast_to(pad[None, :], (NW, EPP - EP))
    sidx = jnp.concatenate([src, pad], axis=1).reshape(NW, NCH, CH)
    didx = jnp.concatenate([dst, pad], axis=1).reshape(NW, NCH, CH)
    xp = jnp.concatenate(
        [features, jnp.zeros((NPAD - N, D), jnp.float32)], axis=0
    )

    sidx_d = sidx.reshape(NW, NPHD, NCHD, CH)
    didx_d = didx.reshape(NW, NPHD, NCHD, CH)
    sidx_s = sidx.reshape(NW, NPH, NCHP, CH)
    didx_s = didx.reshape(NW, NPH, NCHP, CH)
    degp = _deg_call(sidx_d, didx_d).reshape(2 * NC, NPAD)
    hs = _mm_call(xp, W, b.reshape(1, D), degp)
    part = _scatter_call(hs, sidx_s, didx_s)
    return _comb_call(part, degp)[:N]


# speedup vs baseline: 1.3264x; 1.0130x over previous
import functools

import jax
import jax.numpy as jnp
from jax import lax
from jax.experimental import pallas as pl
from jax.experimental.pallas import tpu as pltpu
from jax.experimental.pallas import tpu_sc as plsc

N = 10000
E = 320000
D = 128

NC = 2
NS = 16
NW = NC * NS
NPAD = 10240
EP = E // NW
CH = 64
NB = 5
NCH = 160
EPP = NCH * CH
NPH = 16
NCHP = NCH // NPH
NPHD = 16
NCHD = NCH // NPHD
RPS = NPAD // NS

_MESH = plsc.VectorSubcoreMesh(
    core_axis_name="c", subcore_axis_name="s", num_cores=NC, num_subcores=NS
)


def _deg_body(sidx, didx, degp, sv, dv, ones_v, zv, d0, d1, sem0, sem1, isem):
    c = lax.axis_index("c")
    s = lax.axis_index("s")
    wid = s * NC + c

    def _ones(i, _):
        ones_v[pl.ds(i * 16, 16)] = jnp.ones((16,), jnp.float32)
        return 0

    lax.fori_loop(0, CH // 16, _ones, 0)

    def _zeros(i, _):
        zv[pl.ds(i * 16, 16)] = jnp.zeros((16,), jnp.float32)
        return 0

    lax.fori_loop(0, RPS // 16, _zeros, 0)
    pltpu.sync_copy(zv, d0.at[pl.ds(s * RPS, RPS)])
    pltpu.sync_copy(zv, d1.at[pl.ds(s * RPS, RPS)])
    pltpu.sync_copy(sidx.at[wid, 0], sv.at[0])
    pltpu.sync_copy(didx.at[wid, 0], dv.at[0])
    plsc.subcore_barrier()

    for p in range(NPHD):
        if p + 1 < NPHD:
            pltpu.async_copy(sidx.at[wid, p + 1], sv.at[(p + 1) % 2], isem)
            pltpu.async_copy(didx.at[wid, p + 1], dv.at[(p + 1) % 2], isem)
        svp = sv.at[p % 2]
        dvp = dv.at[p % 2]

        def _scat(j, _, svp=svp, dvp=dvp, first=(p == 0)):
            def _drain():
                pltpu.make_async_copy(ones_v, d0.at[svp.at[0]], sem0).wait()
                pltpu.make_async_copy(ones_v, d1.at[dvp.at[0]], sem1).wait()

            if first:
                pl.when(j > 0)(_drain)
            else:
                _drain()
            pltpu.async_copy(ones_v, d0.at[svp.at[j]], sem0, add=True)
            pltpu.async_copy(ones_v, d1.at[dvp.at[j]], sem1, add=True)
            return 0

        lax.fori_loop(0, NCHD, _scat, 0)
        if p + 1 < NPHD:
            pltpu.make_async_copy(
                sidx.at[wid, p + 1], sv.at[(p + 1) % 2], isem
            ).wait()
            pltpu.make_async_copy(
                didx.at[wid, p + 1], dv.at[(p + 1) % 2], isem
            ).wait()
    pltpu.make_async_copy(ones_v, d0.at[sv.at[0, 0]], sem0).wait()
    pltpu.make_async_copy(ones_v, d1.at[dv.at[0, 0]], sem1).wait()
    plsc.subcore_barrier()
    pltpu.sync_copy(d0.at[pl.ds(s * RPS, RPS)], degp.at[c, 0, pl.ds(s * RPS, RPS)])
    pltpu.sync_copy(d1.at[pl.ds(s * RPS, RPS)], degp.at[c, 1, pl.ds(s * RPS, RPS)])


_deg_call = functools.partial(
    pl.kernel,
    out_type=jax.ShapeDtypeStruct((NC, 2, NPAD), jnp.float32),
    mesh=_MESH,
    scratch_types=[
        pltpu.VMEM((2, NCHD, CH), jnp.int32),
        pltpu.VMEM((2, NCHD, CH), jnp.int32),
        pltpu.VMEM((CH,), jnp.float32),
        pltpu.VMEM((RPS,), jnp.float32),
        pltpu.VMEM_SHARED((NPAD,), jnp.float32),
        pltpu.VMEM_SHARED((NPAD,), jnp.float32),
        pltpu.SemaphoreType.DMA,
        pltpu.SemaphoreType.DMA,
        pltpu.SemaphoreType.DMA,
    ],
)(_deg_body)


def _scatter_body(h, sidx, didx, part, svb, dvb, rows, acc, *sems):
    gsem = sems[:NB]
    ssem = sems[NB : 2 * NB]
    isem = sems[2 * NB]
    c = lax.axis_index("c")
    s = lax.axis_index("s")
    wid = s * NC + c

    def _zb(i, _):
        for k in range(D // 16):
            rows[0, i, pl.ds(k * 16, 16)] = jnp.zeros((16,), jnp.float32)
        return 0

    lax.fori_loop(0, CH, _zb, 0)
    pltpu.sync_copy(sidx.at[wid, 0], svb.at[0])
    pltpu.sync_copy(didx.at[wid, 0], dvb.at[0])
    for k in range(RPS // CH):
        pltpu.sync_copy(rows.at[0], acc.at[pl.ds(s * RPS + k * CH, CH)])
    plsc.subcore_barrier()

    for p in range(NPH):
        if p + 1 < NPH:
            pltpu.async_copy(sidx.at[wid, p + 1], svb.at[(p + 1) % 2], isem)
            pltpu.async_copy(didx.at[wid, p + 1], dvb.at[(p + 1) % 2], isem)
        svp = svb.at[p % 2]
        dvp = dvb.at[p % 2]

        def _group(g, _, svp=svp, dvp=dvp, first=(p == 0)):
            for b in range(NB):

                def _drain(b=b, dvp=dvp):
                    pltpu.make_async_copy(
                        rows.at[b], acc.at[dvp.at[0]], ssem[b]
                    ).wait()

                if first:
                    pl.when(g > 0)(_drain)
                else:
                    _drain()
                pltpu.async_copy(h.at[svp.at[g * NB + b]], rows.at[b], gsem[b])
            for b in range(NB):
                pltpu.make_async_copy(h.at[svp.at[0]], rows.at[b], gsem[b]).wait()
                pltpu.async_copy(
                    rows.at[b], acc.at[dvp.at[g * NB + b]], ssem[b], add=True
                )
            return 0

        lax.fori_loop(0, NCHP // NB, _group, 0)
        if p + 1 < NPH:
            pltpu.make_async_copy(
                sidx.at[wid, p + 1], svb.at[(p + 1) % 2], isem
            ).wait()
            pltpu.make_async_copy(
                didx.at[wid, p + 1], dvb.at[(p + 1) % 2], isem
            ).wait()
    for b in range(NB):
        pltpu.make_async_copy(rows.at[b], acc.at[dvb.at[0, 0]], ssem[b]).wait()
    plsc.subcore_barrier()
    pltpu.sync_copy(acc.at[pl.ds(s * RPS, RPS)], part.at[c, pl.ds(s * RPS, RPS)])


_scatter_call = functools.partial(
    pl.kernel,
    out_type=jax.ShapeDtypeStruct((NC, NPAD, D), jnp.float32),
    mesh=_MESH,
    scratch_types=[
        pltpu.VMEM((2, NCHP, CH), jnp.int32),
        pltpu.VMEM((2, NCHP, CH), jnp.int32),
        pltpu.VMEM((NB, CH, D), jnp.float32),
        pltpu.VMEM_SHARED((NPAD, D), jnp.float32),
    ]
    + [pltpu.SemaphoreType.DMA] * (2 * NB + 1),
)(_scatter_body)


def _mm_body(x_ref, w_ref, b_ref, deg_ref, o_ref):
    h = jnp.dot(x_ref[...], w_ref[...], preferred_element_type=jnp.float32)
    h = h + b_ref[...]
    dsum = deg_ref[0, :] + deg_ref[2, :]
    o_ref[...] = h * lax.rsqrt(jnp.clip(dsum, 1.0, None))[:, None]


_BN1 = 1024
_mm_call = pl.pallas_call(
    _mm_body,
    grid=(NPAD // _BN1,),
    in_specs=[
        pl.BlockSpec((_BN1, D), lambda j: (j, 0)),
        pl.BlockSpec((D, D), lambda j: (0, 0)),
        pl.BlockSpec((1, D), lambda j: (0, 0)),
        pl.BlockSpec((2 * NC, _BN1), lambda j: (0, j)),
    ],
    out_specs=pl.BlockSpec((_BN1, D), lambda j: (j, 0)),
    out_shape=jax.ShapeDtypeStruct((NPAD, D), jnp.float32),
)


def _comb_body(p_ref, deg_ref, o_ref):
    ssum = p_ref[0] + p_ref[1]
    dsum = deg_ref[1, :] + deg_ref[3, :]
    o_ref[...] = ssum * lax.rsqrt(jnp.clip(dsum, 1.0, None))[:, None]


_BN2 = 1024
_comb_call = pl.pallas_call(
    _comb_body,
    grid=(NPAD // _BN2,),
    in_specs=[
        pl.BlockSpec((NC, _BN2, D), lambda j: (0, j, 0)),
        pl.BlockSpec((2 * NC, _BN2), lambda j: (0, j)),
    ],
    out_specs=pl.BlockSpec((_BN2, D), lambda j: (j, 0)),
    out_shape=jax.ShapeDtypeStruct((NPAD, D), jnp.float32),
)


def kernel(features, edge_index, W, b):
    src = edge_index[0].reshape(NW, EP)
    dst = edge_index[1].reshape(NW, EP)
    pad = N + (jnp.arange(EPP - EP, dtype=jnp.int32) % (NPAD - N))
    pad = jnp.broadcast_to(pad[None, :], (NW, EPP - EP))
    sidx = jnp.concatenate([src, pad], axis=1).reshape(NW, NCH, CH)
    didx = jnp.concatenate([dst, pad], axis=1).reshape(NW, NCH, CH)
    xp = jnp.concatenate(
        [features, jnp.zeros((NPAD - N, D), jnp.float32)], axis=0
    )

    sidx_d = sidx.reshape(NW, NPHD, NCHD, CH)
    didx_d = didx.reshape(NW, NPHD, NCHD, CH)
    sidx_s = sidx.reshape(NW, NPH, NCHP, CH)
    didx_s = didx.reshape(NW, NPH, NCHP, CH)
    degp = _deg_call(sidx_d, didx_d).reshape(2 * NC, NPAD)
    hs = _mm_call(xp, W, b.reshape(1, D), degp)
    part = _scatter_call(hs, sidx_s, didx_s)
    return _comb_call(part, degp)[:N]
